# baseline (speedup 1.0000x reference)
import functools

import jax
import jax.numpy as jnp
import numpy as np
from jax import lax
from jax.experimental import pallas as pl
from jax.experimental.pallas import tpu as pltpu
from jax.experimental.pallas import tpu_sc as plsc

N_ACT = 10000
H = 64
F = 2
E_CA = 192000
E_AA = 96000
NUM_CORES = 2
NUM_SUBCORES = 16
NW = NUM_CORES * NUM_SUBCORES

LANE = 128
XC_W = 10112
CA_TILES = E_CA // LANE
AA_TILES = E_AA // LANE
CA_MAIN_T = CA_TILES // NW
AA_MAIN_T = AA_TILES // NW
CA_MAIN = CA_MAIN_T * LANE
AA_MAIN = AA_MAIN_T * LANE
CA_EXTRA = CA_TILES - CA_MAIN_T * NW
AA_EXTRA = AA_TILES - AA_MAIN_T * NW
CA_EXTRA_OFF = CA_MAIN_T * NW * LANE
AA_EXTRA_OFF = AA_MAIN_T * NW * LANE


def _accumulate_edges(e_ref, n_edges, table_ref, acc0, acc1, cnt):
    ones_f = jnp.ones((16,), jnp.float32)
    nfull = n_edges // 16
    assert nfull * 16 == n_edges

    row0 = jnp.zeros((16,), jnp.int32)
    row1 = jnp.ones((16,), jnp.int32)

    @plsc.parallel_loop(0, nfull, unroll=8)
    def _(g):
        s = e_ref[0, pl.ds(g * 16, 16)]
        d = e_ref[1, pl.ds(g * 16, 16)]
        v0 = plsc.load_gather(table_ref, [row0, s])
        v1 = plsc.load_gather(table_ref, [row1, s])
        plsc.addupdate_scatter(acc0, [d], v0)
        plsc.addupdate_scatter(acc1, [d], v1)
        plsc.addupdate_scatter(cnt, [d], ones_f)


def _sc_body(xc_hbm, xa_hbm, eca_hbm, eaa_hbm, out_hbm,
             xc_v, xa_v, eca_v, eaa_v, tca_v, taa_v,
             a_ca0, a_ca1, c_ca, a_aa0, a_aa1, c_aa,
             sem, sem_tca, sem_taa):
    wid = lax.axis_index("s") * NUM_CORES + lax.axis_index("c")

    cp = [
        pltpu.async_copy(xc_hbm.at[:, pl.ds(0, XC_W)], xc_v, sem),
        pltpu.async_copy(xa_hbm, xa_v, sem),
        pltpu.async_copy(eca_hbm.at[:, pl.ds(wid * CA_MAIN, CA_MAIN)],
                         eca_v, sem),
        pltpu.async_copy(eaa_hbm.at[:, pl.ds(wid * AA_MAIN, AA_MAIN)],
                         eaa_v, sem),
    ]

    @pl.when(wid < CA_EXTRA)
    def _():
        pltpu.async_copy(
            eca_hbm.at[:, pl.ds(CA_EXTRA_OFF + wid * LANE, LANE)],
            tca_v, sem_tca)

    @pl.when(wid < AA_EXTRA)
    def _():
        pltpu.async_copy(
            eaa_hbm.at[:, pl.ds(AA_EXTRA_OFF + wid * LANE, LANE)],
            taa_v, sem_taa)

    zf = jnp.zeros((16,), jnp.float32)

    @plsc.parallel_loop(0, N_ACT // 16, unroll=8)
    def _(i):
        for r in (a_ca0, a_ca1, c_ca, a_aa0, a_aa1, c_aa):
            r[pl.ds(i * 16, 16)] = zf

    for c in cp:
        c.wait()

    _accumulate_edges(eca_v, CA_MAIN, xc_v, a_ca0, a_ca1, c_ca)

    @pl.when(wid < CA_EXTRA)
    def _():
        pltpu.make_async_copy(
            eca_hbm.at[:, pl.ds(CA_EXTRA_OFF + wid * LANE, LANE)],
            tca_v, sem_tca).wait()
        _accumulate_edges(tca_v, LANE, xc_v, a_ca0, a_ca1, c_ca)

    ca_cp = [
        pltpu.async_copy(r, out_hbm.at[j, wid], sem)
        for j, r in enumerate((a_ca0, a_ca1, c_ca))
    ]

    _accumulate_edges(eaa_v, AA_MAIN, xa_v, a_aa0, a_aa1, c_aa)

    @pl.when(wid < AA_EXTRA)
    def _():
        pltpu.make_async_copy(
            eaa_hbm.at[:, pl.ds(AA_EXTRA_OFF + wid * LANE, LANE)],
            taa_v, sem_taa).wait()
        _accumulate_edges(taa_v, LANE, xa_v, a_aa0, a_aa1, c_aa)

    out_cp = ca_cp + [
        pltpu.async_copy(r, out_hbm.at[j + 3, wid], sem)
        for j, r in enumerate((a_aa0, a_aa1, c_aa))
    ]
    for c in out_cp:
        c.wait()


def _sc_partials(xcT, xaT, eca, eaa):
    mesh = plsc.VectorSubcoreMesh(core_axis_name="c", subcore_axis_name="s",
                                  num_cores=NUM_CORES, num_subcores=NUM_SUBCORES)
    fn = pl.kernel(
        _sc_body,
        out_type=jax.ShapeDtypeStruct((6, NW, N_ACT), jnp.float32),
        mesh=mesh,
        compiler_params=pltpu.CompilerParams(needs_layout_passes=False),
        scratch_types=[
            pltpu.VMEM((F, XC_W), jnp.float32),
            pltpu.VMEM((F, N_ACT), jnp.float32),
            pltpu.VMEM((2, CA_MAIN), jnp.int32),
            pltpu.VMEM((2, AA_MAIN), jnp.int32),
            pltpu.VMEM((2, LANE), jnp.int32),
            pltpu.VMEM((2, LANE), jnp.int32),
            pltpu.VMEM((N_ACT,), jnp.float32),
            pltpu.VMEM((N_ACT,), jnp.float32),
            pltpu.VMEM((N_ACT,), jnp.float32),
            pltpu.VMEM((N_ACT,), jnp.float32),
            pltpu.VMEM((N_ACT,), jnp.float32),
            pltpu.VMEM((N_ACT,), jnp.float32),
            pltpu.SemaphoreType.DMA,
            pltpu.SemaphoreType.DMA,
            pltpu.SemaphoreType.DMA,
        ],
        name="hetero_sage_segment_sums",
    )
    return fn(xcT, xaT, eca, eaa)


_LOG1E4 = float(np.log(10000.0) / (H // 2 - 1))
_NBLK = 4
_BW = NW // _NBLK


def _tc_body(S_ref, xaT_ref, ts_ref,
             Wce_ref, bce_ref, Wae_ref, bae_ref,
             Wt1_ref, bt1_ref, Wt2T_ref, bt2_ref,
             Wrca_ref, Wnca_ref, bca_ref,
             Wraa_ref, Wnaa_ref, baa_ref,
             Wp1T_ref, bp1_ref, Wp2T_ref, bp2_ref,
             out_ref, P_acc):
    i = pl.program_id(0)
    blk = jnp.sum(S_ref[...], axis=1)

    @pl.when(i == 0)
    def _():
        P_acc[...] = blk

    @pl.when(i > 0)
    def _():
        P_acc[...] += blk

    @pl.when(i == _NBLK - 1)
    def _():
        dg = functools.partial(lax.dot_general,
                               precision=lax.Precision.HIGHEST,
                               preferred_element_type=jnp.float32)
        cdims = (((0,), (0,)), ((), ()))
        tdims = (((0,), (1,)), ((), ()))
        rdims = (((1,), (0,)), ((), ()))

        P = P_acc[...]
        n_ca = P[2:3]
        n_aa = P[5:6]
        inv_ca = 1.0 / jnp.maximum(n_ca, 1.0)
        inv_aa = 1.0 / jnp.maximum(n_aa, 1.0)
        V = jnp.concatenate([
            P[0:1] * inv_ca, P[1:2] * inv_ca,
            P[3:4] * inv_aa, P[4:5] * inv_aa,
            xaT_ref[...],
            (n_ca > 0).astype(jnp.float32),
            (n_aa > 0).astype(jnp.float32),
        ], axis=0)

        Wce = Wce_ref[...]
        Wae = Wae_ref[...]
        Wnca = Wnca_ref[...]
        Wnaa = Wnaa_ref[...]
        Wr_sum = Wrca_ref[...] + Wraa_ref[...]
        A_ca = dg(Wnca, Wce, tdims)
        A_aa = dg(Wnaa, Wae, tdims)
        RT = dg(Wr_sum, Wae, tdims)

        bce_row = dg(bce_ref[...][None], Wnca, rdims)
        bae_row = dg(bae_ref[...][None], Wnaa, rdims)
        bconst_row = (bca_ref[...] + baa_ref[...])[None] \
            + dg(bae_ref[...][None], Wr_sum, rdims)

        t = ts_ref[0].astype(jnp.float32)
        freqs = jnp.exp(lax.broadcasted_iota(jnp.int32, (1, H // 2), 1)
                        .astype(jnp.float32) * (-_LOG1E4))
        args = freqs * t
        te = jnp.concatenate([jnp.sin(args), jnp.cos(args)], axis=1)
        h_t = dg(te, Wt1_ref[...], rdims) + bt1_ref[...][None]
        te2 = dg(jax.nn.gelu(h_t), Wt2T_ref[...],
                 (((1,), (1,)), ((), ()))) + bt2_ref[...][None]

        Wp1T = Wp1T_ref[...]
        tec_row = dg(te2, Wp1T[:, H:],
                     (((1,), (1,)), ((), ()))) + bp1_ref[...][None]

        Z = jnp.concatenate([tec_row, bconst_row, bce_row, bae_row],
                            axis=0)
        ZT = Z.T
        te_contrib = ZT[:, 0:1]
        b_const = ZT[:, 1:2]
        bceT = ZT[:, 2:3]
        baeT = ZT[:, 3:4]

        W_all = jnp.concatenate([A_ca, A_aa, RT, bceT, baeT], axis=1)
        out_aT = dg(W_all, V, rdims) + b_const
        actT = jnp.maximum(out_aT, 0.0)
        h1T = dg(Wp1T[:, :H], actT, rdims) + te_contrib
        gT = jax.nn.gelu(h1T)
        out_ref[...] = dg(Wp2T_ref[...], gT, rdims) \
            + bp2_ref[...][:, None]


def _tc_epilogue(S, xaT, timestep, Wce, bce, Wae, bae, Wt1, bt1, Wt2, bt2,
                 Wrca, Wnca, bca, Wraa, Wnaa, baa, Wp1, bp1, Wp2, bp2):
    def whole(a):
        return pl.BlockSpec(a.shape, lambda i: (0,) * a.ndim)

    in_specs = [
        pl.BlockSpec((6, _BW, N_ACT), lambda i: (0, i, 0)),
        whole(xaT),
        pl.BlockSpec(memory_space=pltpu.SMEM),
    ] + [whole(a) for a in (Wce, bce, Wae, bae, Wt1, bt1, Wt2, bt2,
                            Wrca, Wnca, bca, Wraa, Wnaa, baa,
                            Wp1, bp1, Wp2, bp2)]
    return pl.pallas_call(
        _tc_body,
        grid=(_NBLK,),
        in_specs=in_specs,
        out_specs=pl.BlockSpec((F, N_ACT), lambda i: (0, 0)),
        out_shape=jax.ShapeDtypeStruct((F, N_ACT), jnp.float32),
        scratch_shapes=[pltpu.VMEM((6, N_ACT), jnp.float32)],
    )(S, xaT, timestep, Wce, bce, Wae, bae, Wt1, bt1, Wt2, bt2,
      Wrca, Wnca, bca, Wraa, Wnaa, baa, Wp1, bp1, Wp2, bp2)


def kernel(x_context, x_action, edge_index_cc, edge_index_ca, edge_index_aa,
           timestep, W_ce, b_ce, W_ae, b_ae, Wt1, bt1, Wt2, bt2,
           Wr_cc, Wn_cc, b_cc, Wr_ca, Wn_ca, b_ca, Wr_aa, Wn_aa, b_aa,
           Wp1, bp1, Wp2, bp2):
    del edge_index_cc, Wr_cc, Wn_cc, b_cc
    S = _sc_partials(x_context.T, x_action.T, edge_index_ca, edge_index_aa)
    predT = _tc_epilogue(
        S, x_action.T, timestep,
        W_ce, b_ce, W_ae, b_ae, Wt1, bt1, Wt2.T, bt2,
        Wr_ca, Wn_ca, b_ca, Wr_aa, Wn_aa, b_aa, Wp1.T, bp1, Wp2.T, bp2)
    return predT.T


# speedup vs baseline: 1.0066x; 1.0066x over previous
import functools

import jax
import jax.numpy as jnp
import numpy as np
from jax import lax
from jax.experimental import pallas as pl
from jax.experimental.pallas import tpu as pltpu
from jax.experimental.pallas import tpu_sc as plsc

N_ACT = 10000
H = 64
F = 2
E_CA = 192000
E_AA = 96000
NUM_CORES = 2
NUM_SUBCORES = 16
NW = NUM_CORES * NUM_SUBCORES

LANE = 128
XC_W = 10112
CA_TILES = E_CA // LANE
AA_TILES = E_AA // LANE
CA_MAIN_T = CA_TILES // NW
AA_MAIN_T = AA_TILES // NW
CA_MAIN = CA_MAIN_T * LANE
AA_MAIN = AA_MAIN_T * LANE
CA_EXTRA = CA_TILES - CA_MAIN_T * NW
AA_EXTRA = AA_TILES - AA_MAIN_T * NW
CA_EXTRA_OFF = CA_MAIN_T * NW * LANE
AA_EXTRA_OFF = AA_MAIN_T * NW * LANE


def _accumulate_edges(e_ref, n_edges, table_ref, acc0, acc1, cnt):
    ones_f = jnp.ones((16,), jnp.float32)
    nfull = n_edges // 16
    assert nfull * 16 == n_edges

    row0 = jnp.zeros((16,), jnp.int32)
    row1 = jnp.ones((16,), jnp.int32)

    @plsc.parallel_loop(0, nfull, unroll=4)
    def _(g):
        s = e_ref[0, pl.ds(g * 16, 16)]
        d = e_ref[1, pl.ds(g * 16, 16)]
        v0 = plsc.load_gather(table_ref, [row0, s])
        v1 = plsc.load_gather(table_ref, [row1, s])
        plsc.addupdate_scatter(acc0, [d], v0)
        plsc.addupdate_scatter(acc1, [d], v1)
        plsc.addupdate_scatter(cnt, [d], ones_f)


def _sc_body(xc_hbm, xa_hbm, eca_hbm, eaa_hbm, out_hbm,
             xc_v, xa_v, eca_v, eaa_v, tca_v, taa_v,
             a_ca0, a_ca1, c_ca, a_aa0, a_aa1, c_aa,
             sem, sem_tca, sem_taa):
    wid = lax.axis_index("s") * NUM_CORES + lax.axis_index("c")

    cp = [
        pltpu.async_copy(xc_hbm.at[:, pl.ds(0, XC_W)], xc_v, sem),
        pltpu.async_copy(xa_hbm, xa_v, sem),
        pltpu.async_copy(eca_hbm.at[:, pl.ds(wid * CA_MAIN, CA_MAIN)],
                         eca_v, sem),
        pltpu.async_copy(eaa_hbm.at[:, pl.ds(wid * AA_MAIN, AA_MAIN)],
                         eaa_v, sem),
    ]

    @pl.when(wid < CA_EXTRA)
    def _():
        pltpu.async_copy(
            eca_hbm.at[:, pl.ds(CA_EXTRA_OFF + wid * LANE, LANE)],
            tca_v, sem_tca)

    @pl.when(wid < AA_EXTRA)
    def _():
        pltpu.async_copy(
            eaa_hbm.at[:, pl.ds(AA_EXTRA_OFF + wid * LANE, LANE)],
            taa_v, sem_taa)

    zf = jnp.zeros((16,), jnp.float32)

    @plsc.parallel_loop(0, N_ACT // 16, unroll=4)
    def _(i):
        for r in (a_ca0, a_ca1, c_ca, a_aa0, a_aa1, c_aa):
            r[pl.ds(i * 16, 16)] = zf

    for c in cp:
        c.wait()

    _accumulate_edges(eca_v, CA_MAIN, xc_v, a_ca0, a_ca1, c_ca)

    @pl.when(wid < CA_EXTRA)
    def _():
        pltpu.make_async_copy(
            eca_hbm.at[:, pl.ds(CA_EXTRA_OFF + wid * LANE, LANE)],
            tca_v, sem_tca).wait()
        _accumulate_edges(tca_v, LANE, xc_v, a_ca0, a_ca1, c_ca)

    ca_cp = [
        pltpu.async_copy(r, out_hbm.at[j, wid], sem)
        for j, r in enumerate((a_ca0, a_ca1, c_ca))
    ]

    _accumulate_edges(eaa_v, AA_MAIN, xa_v, a_aa0, a_aa1, c_aa)

    @pl.when(wid < AA_EXTRA)
    def _():
        pltpu.make_async_copy(
            eaa_hbm.at[:, pl.ds(AA_EXTRA_OFF + wid * LANE, LANE)],
            taa_v, sem_taa).wait()
        _accumulate_edges(taa_v, LANE, xa_v, a_aa0, a_aa1, c_aa)

    out_cp = ca_cp + [
        pltpu.async_copy(r, out_hbm.at[j + 3, wid], sem)
        for j, r in enumerate((a_aa0, a_aa1, c_aa))
    ]
    for c in out_cp:
        c.wait()


def _sc_partials(xcT, xaT, eca, eaa):
    mesh = plsc.VectorSubcoreMesh(core_axis_name="c", subcore_axis_name="s",
                                  num_cores=NUM_CORES, num_subcores=NUM_SUBCORES)
    fn = pl.kernel(
        _sc_body,
        out_type=jax.ShapeDtypeStruct((6, NW, N_ACT), jnp.float32),
        mesh=mesh,
        compiler_params=pltpu.CompilerParams(needs_layout_passes=False),
        scratch_types=[
            pltpu.VMEM((F, XC_W), jnp.float32),
            pltpu.VMEM((F, N_ACT), jnp.float32),
            pltpu.VMEM((2, CA_MAIN), jnp.int32),
            pltpu.VMEM((2, AA_MAIN), jnp.int32),
            pltpu.VMEM((2, LANE), jnp.int32),
            pltpu.VMEM((2, LANE), jnp.int32),
            pltpu.VMEM((N_ACT,), jnp.float32),
            pltpu.VMEM((N_ACT,), jnp.float32),
            pltpu.VMEM((N_ACT,), jnp.float32),
            pltpu.VMEM((N_ACT,), jnp.float32),
            pltpu.VMEM((N_ACT,), jnp.float32),
            pltpu.VMEM((N_ACT,), jnp.float32),
            pltpu.SemaphoreType.DMA,
            pltpu.SemaphoreType.DMA,
            pltpu.SemaphoreType.DMA,
        ],
        name="hetero_sage_segment_sums",
    )
    return fn(xcT, xaT, eca, eaa)


_LOG1E4 = float(np.log(10000.0) / (H // 2 - 1))
_NBLK = 4
_BW = NW // _NBLK


def _tc_body(S_ref, xaT_ref, ts_ref,
             Wce_ref, bce_ref, Wae_ref, bae_ref,
             Wt1_ref, bt1_ref, Wt2T_ref, bt2_ref,
             Wrca_ref, Wnca_ref, bca_ref,
             Wraa_ref, Wnaa_ref, baa_ref,
             Wp1T_ref, bp1_ref, Wp2T_ref, bp2_ref,
             out_ref, P_acc):
    i = pl.program_id(0)
    blk = jnp.sum(S_ref[...], axis=1)

    @pl.when(i == 0)
    def _():
        P_acc[...] = blk

    @pl.when(i > 0)
    def _():
        P_acc[...] += blk

    @pl.when(i == _NBLK - 1)
    def _():
        dg = functools.partial(lax.dot_general,
                               precision=lax.Precision.HIGHEST,
                               preferred_element_type=jnp.float32)
        cdims = (((0,), (0,)), ((), ()))
        tdims = (((0,), (1,)), ((), ()))
        rdims = (((1,), (0,)), ((), ()))

        P = P_acc[...]
        n_ca = P[2:3]
        n_aa = P[5:6]
        inv_ca = 1.0 / jnp.maximum(n_ca, 1.0)
        inv_aa = 1.0 / jnp.maximum(n_aa, 1.0)
        V = jnp.concatenate([
            P[0:1] * inv_ca, P[1:2] * inv_ca,
            P[3:4] * inv_aa, P[4:5] * inv_aa,
            xaT_ref[...],
            (n_ca > 0).astype(jnp.float32),
            (n_aa > 0).astype(jnp.float32),
        ], axis=0)

        Wce = Wce_ref[...]
        Wae = Wae_ref[...]
        Wnca = Wnca_ref[...]
        Wnaa = Wnaa_ref[...]
        Wr_sum = Wrca_ref[...] + Wraa_ref[...]
        A_ca = dg(Wnca, Wce, tdims)
        A_aa = dg(Wnaa, Wae, tdims)
        RT = dg(Wr_sum, Wae, tdims)

        bce_row = dg(bce_ref[...][None], Wnca, rdims)
        bae_row = dg(bae_ref[...][None], Wnaa, rdims)
        bconst_row = (bca_ref[...] + baa_ref[...])[None] \
            + dg(bae_ref[...][None], Wr_sum, rdims)

        t = ts_ref[0].astype(jnp.float32)
        freqs = jnp.exp(lax.broadcasted_iota(jnp.int32, (1, H // 2), 1)
                        .astype(jnp.float32) * (-_LOG1E4))
        args = freqs * t
        te = jnp.concatenate([jnp.sin(args), jnp.cos(args)], axis=1)
        h_t = dg(te, Wt1_ref[...], rdims) + bt1_ref[...][None]
        te2 = dg(jax.nn.gelu(h_t), Wt2T_ref[...],
                 (((1,), (1,)), ((), ()))) + bt2_ref[...][None]

        Wp1T = Wp1T_ref[...]
        tec_row = dg(te2, Wp1T[:, H:],
                     (((1,), (1,)), ((), ()))) + bp1_ref[...][None]

        Z = jnp.concatenate([tec_row, bconst_row, bce_row, bae_row],
                            axis=0)
        ZT = Z.T
        te_contrib = ZT[:, 0:1]
        b_const = ZT[:, 1:2]
        bceT = ZT[:, 2:3]
        baeT = ZT[:, 3:4]

        W_all = jnp.concatenate([A_ca, A_aa, RT, bceT, baeT], axis=1)
        out_aT = dg(W_all, V, rdims) + b_const
        actT = jnp.maximum(out_aT, 0.0)
        h1T = dg(Wp1T[:, :H], actT, rdims) + te_contrib
        gT = jax.nn.gelu(h1T)
        out_ref[...] = dg(Wp2T_ref[...], gT, rdims) \
            + bp2_ref[...][:, None]


def _tc_epilogue(S, xaT, timestep, Wce, bce, Wae, bae, Wt1, bt1, Wt2, bt2,
                 Wrca, Wnca, bca, Wraa, Wnaa, baa, Wp1, bp1, Wp2, bp2):
    def whole(a):
        return pl.BlockSpec(a.shape, lambda i: (0,) * a.ndim)

    in_specs = [
        pl.BlockSpec((6, _BW, N_ACT), lambda i: (0, i, 0)),
        whole(xaT),
        pl.BlockSpec(memory_space=pltpu.SMEM),
    ] + [whole(a) for a in (Wce, bce, Wae, bae, Wt1, bt1, Wt2, bt2,
                            Wrca, Wnca, bca, Wraa, Wnaa, baa,
                            Wp1, bp1, Wp2, bp2)]
    return pl.pallas_call(
        _tc_body,
        grid=(_NBLK,),
        in_specs=in_specs,
        out_specs=pl.BlockSpec((F, N_ACT), lambda i: (0, 0)),
        out_shape=jax.ShapeDtypeStruct((F, N_ACT), jnp.float32),
        scratch_shapes=[pltpu.VMEM((6, N_ACT), jnp.float32)],
    )(S, xaT, timestep, Wce, bce, Wae, bae, Wt1, bt1, Wt2, bt2,
      Wrca, Wnca, bca, Wraa, Wnaa, baa, Wp1, bp1, Wp2, bp2)


def kernel(x_context, x_action, edge_index_cc, edge_index_ca, edge_index_aa,
           timestep, W_ce, b_ce, W_ae, b_ae, Wt1, bt1, Wt2, bt2,
           Wr_cc, Wn_cc, b_cc, Wr_ca, Wn_ca, b_ca, Wr_aa, Wn_aa, b_aa,
           Wp1, bp1, Wp2, bp2):
    del edge_index_cc, Wr_cc, Wn_cc, b_cc
    S = _sc_partials(x_context.T, x_action.T, edge_index_ca, edge_index_aa)
    predT = _tc_epilogue(
        S, x_action.T, timestep,
        W_ce, b_ce, W_ae, b_ae, Wt1, bt1, Wt2.T, bt2,
        Wr_ca, Wn_ca, b_ca, Wr_aa, Wn_aa, b_aa, Wp1.T, bp1, Wp2.T, bp2)
    return predT.T


# speedup vs baseline: 1.0581x; 1.0511x over previous
import functools

import jax
import jax.numpy as jnp
import numpy as np
from jax import lax
from jax.experimental import pallas as pl
from jax.experimental.pallas import tpu as pltpu
from jax.experimental.pallas import tpu_sc as plsc

N_ACT = 10000
H = 64
F = 2
E_CA = 192000
E_AA = 96000
NUM_CORES = 2
NUM_SUBCORES = 16
NW = NUM_CORES * NUM_SUBCORES

LANE = 128
XC_W = 10112
CA_TILES = E_CA // LANE
AA_TILES = E_AA // LANE
CA_MAIN_T = CA_TILES // NW
AA_MAIN_T = AA_TILES // NW
CA_MAIN = CA_MAIN_T * LANE
AA_MAIN = AA_MAIN_T * LANE
CA_EXTRA = CA_TILES - CA_MAIN_T * NW
AA_EXTRA = AA_TILES - AA_MAIN_T * NW
CA_EXTRA_OFF = CA_MAIN_T * NW * LANE
AA_EXTRA_OFF = AA_MAIN_T * NW * LANE


def _accumulate_edges(e_ref, n_edges, table_ref, acc0, acc1, cnt):
    ones_f = jnp.ones((16,), jnp.float32)
    nfull = n_edges // 16
    assert nfull * 16 == n_edges

    row0 = jnp.zeros((16,), jnp.int32)
    row1 = jnp.ones((16,), jnp.int32)

    @plsc.parallel_loop(0, nfull, unroll=4)
    def _(g):
        s = e_ref[0, pl.ds(g * 16, 16)]
        d = e_ref[1, pl.ds(g * 16, 16)]
        v0 = plsc.load_gather(table_ref, [row0, s])
        v1 = plsc.load_gather(table_ref, [row1, s])
        plsc.addupdate_scatter(acc0, [d], v0)
        plsc.addupdate_scatter(acc1, [d], v1)
        plsc.addupdate_scatter(cnt, [d], ones_f)


def _sc_body(xc_hbm, xa_hbm, eca_hbm, eaa_hbm, out_hbm,
             xc_v, xa_v, eca_v, eaa_v, tca_v, taa_v,
             a_ca0, a_ca1, c_ca, a_aa0, a_aa1, c_aa,
             sem, sem_tca, sem_taa):
    wid = lax.axis_index("s") * NUM_CORES + lax.axis_index("c")

    cp = [
        pltpu.async_copy(xc_hbm.at[:, pl.ds(0, XC_W)], xc_v, sem),
        pltpu.async_copy(xa_hbm, xa_v, sem),
        pltpu.async_copy(eca_hbm.at[:, pl.ds(wid * CA_MAIN, CA_MAIN)],
                         eca_v, sem),
        pltpu.async_copy(eaa_hbm.at[:, pl.ds(wid * AA_MAIN, AA_MAIN)],
                         eaa_v, sem),
    ]

    @pl.when(wid < CA_EXTRA)
    def _():
        pltpu.async_copy(
            eca_hbm.at[:, pl.ds(CA_EXTRA_OFF + wid * LANE, LANE)],
            tca_v, sem_tca)

    @pl.when(wid < AA_EXTRA)
    def _():
        pltpu.async_copy(
            eaa_hbm.at[:, pl.ds(AA_EXTRA_OFF + wid * LANE, LANE)],
            taa_v, sem_taa)

    zf = jnp.zeros((16,), jnp.float32)

    @plsc.parallel_loop(0, N_ACT // 16, unroll=4)
    def _(i):
        for r in (a_ca0, a_ca1, c_ca, a_aa0, a_aa1, c_aa):
            r[pl.ds(i * 16, 16)] = zf

    for c in cp:
        c.wait()

    _accumulate_edges(eca_v, CA_MAIN, xc_v, a_ca0, a_ca1, c_ca)

    @pl.when(wid < CA_EXTRA)
    def _():
        pltpu.make_async_copy(
            eca_hbm.at[:, pl.ds(CA_EXTRA_OFF + wid * LANE, LANE)],
            tca_v, sem_tca).wait()
        _accumulate_edges(tca_v, LANE, xc_v, a_ca0, a_ca1, c_ca)

    ca_cp = [
        pltpu.async_copy(r, out_hbm.at[j, wid], sem)
        for j, r in enumerate((a_ca0, a_ca1, c_ca))
    ]

    _accumulate_edges(eaa_v, AA_MAIN, xa_v, a_aa0, a_aa1, c_aa)

    @pl.when(wid < AA_EXTRA)
    def _():
        pltpu.make_async_copy(
            eaa_hbm.at[:, pl.ds(AA_EXTRA_OFF + wid * LANE, LANE)],
            taa_v, sem_taa).wait()
        _accumulate_edges(taa_v, LANE, xa_v, a_aa0, a_aa1, c_aa)

    out_cp = ca_cp + [
        pltpu.async_copy(r, out_hbm.at[j + 3, wid], sem)
        for j, r in enumerate((a_aa0, a_aa1, c_aa))
    ]
    for c in out_cp:
        c.wait()


def _sc_partials(xcT, xaT, eca, eaa):
    mesh = plsc.VectorSubcoreMesh(core_axis_name="c", subcore_axis_name="s",
                                  num_cores=NUM_CORES, num_subcores=NUM_SUBCORES)
    fn = pl.kernel(
        _sc_body,
        out_type=jax.ShapeDtypeStruct((6, NW, N_ACT), jnp.float32),
        mesh=mesh,
        compiler_params=pltpu.CompilerParams(needs_layout_passes=False),
        scratch_types=[
            pltpu.VMEM((F, XC_W), jnp.float32),
            pltpu.VMEM((F, N_ACT), jnp.float32),
            pltpu.VMEM((2, CA_MAIN), jnp.int32),
            pltpu.VMEM((2, AA_MAIN), jnp.int32),
            pltpu.VMEM((2, LANE), jnp.int32),
            pltpu.VMEM((2, LANE), jnp.int32),
            pltpu.VMEM((N_ACT,), jnp.float32),
            pltpu.VMEM((N_ACT,), jnp.float32),
            pltpu.VMEM((N_ACT,), jnp.float32),
            pltpu.VMEM((N_ACT,), jnp.float32),
            pltpu.VMEM((N_ACT,), jnp.float32),
            pltpu.VMEM((N_ACT,), jnp.float32),
            pltpu.SemaphoreType.DMA,
            pltpu.SemaphoreType.DMA,
            pltpu.SemaphoreType.DMA,
        ],
        name="hetero_sage_segment_sums",
    )
    return fn(xcT, xaT, eca, eaa)


_LOG1E4 = float(np.log(10000.0) / (H // 2 - 1))
_NBLK = 4
_BW = NW // _NBLK


def _tc_body(S_ref, xaT_ref, ts_ref,
             Wce_ref, bce_ref, Wae_ref, bae_ref,
             Wt1_ref, bt1_ref, Wt2T_ref, bt2_ref,
             Wrca_ref, Wnca_ref, bca_ref,
             Wraa_ref, Wnaa_ref, baa_ref,
             Wp1T_ref, bp1_ref, Wp2T_ref, bp2_ref,
             out_ref, P_acc):
    i = pl.program_id(0)
    blk = jnp.sum(S_ref[...], axis=1)

    @pl.when(i == 0)
    def _():
        P_acc[...] = blk

    @pl.when(i > 0)
    def _():
        P_acc[...] += blk

    @pl.when(i == _NBLK - 1)
    def _():
        dg = functools.partial(lax.dot_general,
                               precision=lax.Precision.HIGHEST,
                               preferred_element_type=jnp.float32)
        dd = functools.partial(lax.dot_general,
                               precision=lax.Precision.DEFAULT,
                               preferred_element_type=jnp.float32)
        cdims = (((0,), (0,)), ((), ()))
        tdims = (((0,), (1,)), ((), ()))
        rdims = (((1,), (0,)), ((), ()))

        P = P_acc[...]
        n_ca = P[2:3]
        n_aa = P[5:6]
        inv_ca = 1.0 / jnp.maximum(n_ca, 1.0)
        inv_aa = 1.0 / jnp.maximum(n_aa, 1.0)
        V = jnp.concatenate([
            P[0:1] * inv_ca, P[1:2] * inv_ca,
            P[3:4] * inv_aa, P[4:5] * inv_aa,
            xaT_ref[...],
            (n_ca > 0).astype(jnp.float32),
            (n_aa > 0).astype(jnp.float32),
        ], axis=0)

        Wce = Wce_ref[...]
        Wae = Wae_ref[...]
        Wnca = Wnca_ref[...]
        Wnaa = Wnaa_ref[...]
        Wr_sum = Wrca_ref[...] + Wraa_ref[...]
        A_ca = dg(Wnca, Wce, tdims)
        A_aa = dg(Wnaa, Wae, tdims)
        RT = dg(Wr_sum, Wae, tdims)

        bce_row = dg(bce_ref[...][None], Wnca, rdims)
        bae_row = dg(bae_ref[...][None], Wnaa, rdims)
        bconst_row = (bca_ref[...] + baa_ref[...])[None] \
            + dg(bae_ref[...][None], Wr_sum, rdims)

        t = ts_ref[0].astype(jnp.float32)
        freqs = jnp.exp(lax.broadcasted_iota(jnp.int32, (1, H // 2), 1)
                        .astype(jnp.float32) * (-_LOG1E4))
        args = freqs * t
        te = jnp.concatenate([jnp.sin(args), jnp.cos(args)], axis=1)
        h_t = dd(te, Wt1_ref[...], rdims) + bt1_ref[...][None]
        te2 = dd(jax.nn.gelu(h_t), Wt2T_ref[...],
                 (((1,), (1,)), ((), ()))) + bt2_ref[...][None]

        Wp1T = Wp1T_ref[...]
        tec_row = dd(te2, Wp1T[:, H:],
                     (((1,), (1,)), ((), ()))) + bp1_ref[...][None]

        Z = jnp.concatenate([tec_row, bconst_row, bce_row, bae_row],
                            axis=0)
        ZT = Z.T
        te_contrib = ZT[:, 0:1]
        b_const = ZT[:, 1:2]
        bceT = ZT[:, 2:3]
        baeT = ZT[:, 3:4]

        W_all = jnp.concatenate([A_ca, A_aa, RT, bceT, baeT], axis=1)
        out_aT = dg(W_all, V, rdims) + b_const
        actT = jnp.maximum(out_aT, 0.0)
        h1T = dd(Wp1T[:, :H], actT, rdims) + te_contrib
        gT = jax.nn.gelu(h1T)
        out_ref[...] = dd(Wp2T_ref[...], gT, rdims) \
            + bp2_ref[...][:, None]


def _tc_epilogue(S, xaT, timestep, Wce, bce, Wae, bae, Wt1, bt1, Wt2, bt2,
                 Wrca, Wnca, bca, Wraa, Wnaa, baa, Wp1, bp1, Wp2, bp2):
    def whole(a):
        return pl.BlockSpec(a.shape, lambda i: (0,) * a.ndim)

    in_specs = [
        pl.BlockSpec((6, _BW, N_ACT), lambda i: (0, i, 0)),
        whole(xaT),
        pl.BlockSpec(memory_space=pltpu.SMEM),
    ] + [whole(a) for a in (Wce, bce, Wae, bae, Wt1, bt1, Wt2, bt2,
                            Wrca, Wnca, bca, Wraa, Wnaa, baa,
                            Wp1, bp1, Wp2, bp2)]
    return pl.pallas_call(
        _tc_body,
        grid=(_NBLK,),
        in_specs=in_specs,
        out_specs=pl.BlockSpec((F, N_ACT), lambda i: (0, 0)),
        out_shape=jax.ShapeDtypeStruct((F, N_ACT), jnp.float32),
        scratch_shapes=[pltpu.VMEM((6, N_ACT), jnp.float32)],
    )(S, xaT, timestep, Wce, bce, Wae, bae, Wt1, bt1, Wt2, bt2,
      Wrca, Wnca, bca, Wraa, Wnaa, baa, Wp1, bp1, Wp2, bp2)


def kernel(x_context, x_action, edge_index_cc, edge_index_ca, edge_index_aa,
           timestep, W_ce, b_ce, W_ae, b_ae, Wt1, bt1, Wt2, bt2,
           Wr_cc, Wn_cc, b_cc, Wr_ca, Wn_ca, b_ca, Wr_aa, Wn_aa, b_aa,
           Wp1, bp1, Wp2, bp2):
    del edge_index_cc, Wr_cc, Wn_cc, b_cc
    S = _sc_partials(x_context.T, x_action.T, edge_index_ca, edge_index_aa)
    predT = _tc_epilogue(
        S, x_action.T, timestep,
        W_ce, b_ce, W_ae, b_ae, Wt1, bt1, Wt2.T, bt2,
        Wr_ca, Wn_ca, b_ca, Wr_aa, Wn_aa, b_aa, Wp1.T, bp1, Wp2.T, bp2)
    return predT.T
